# baseline (speedup 1.0000x reference)
import functools

import jax
import jax.numpy as jnp
from jax import lax
from jax.experimental import pallas as pl
from jax.experimental.pallas import tpu as pltpu
from jax.experimental.pallas import tpu_sc as plsc

N = 10000
D = 128
E = 320000

NC = 2
NS = 16
NW = NC * NS

CHUNK = 128
MAIN_ROWS = 2496
TAIL = (E // CHUNK - MAIN_ROWS) * CHUNK
PASS_MAX = 40

DUMMY_ROWS = 64
ACC_ROWS = N + DUMMY_ROWS
ZROWS = 632
COPY_ROWS = 632


def _sc_agg_build():
  mesh = plsc.VectorSubcoreMesh(core_axis_name="c", subcore_axis_name="s")

  @functools.partial(
      pl.kernel,
      mesh=mesh,
      out_type=jax.ShapeDtypeStruct((NC, N, D), jnp.float32),
      scratch_types=[
          pltpu.VMEM((48, CHUNK), jnp.int32),
          pltpu.VMEM((48, CHUNK), jnp.int32),
          pltpu.VMEM((CHUNK, D), jnp.float32),
          pltpu.VMEM((CHUNK, D), jnp.float32),
          pltpu.VMEM_SHARED((ACC_ROWS, D), jnp.float32),
          pltpu.SemaphoreType.DMA,
          pltpu.SemaphoreType.DMA,
          pltpu.SemaphoreType.DMA,
          pltpu.SemaphoreType.DMA,
      ],
  )
  def sc_agg(x_hbm, src_hbm, dst_hbm, tail_hbm, out_hbm, srcb, dstb, rows0,
             rows1, acc, sem0, sem1, semA, semB):
    cid = lax.axis_index("c")
    sid = lax.axis_index("s")
    wid = cid * NS + sid

    def idx_load(row_base, c, reg):
      pltpu.async_copy(src_hbm.at[pl.ds(row_base, c)],
                       srcb.at[pl.ds(reg * 24, c)], semA)
      pltpu.async_copy(dst_hbm.at[pl.ds(row_base, c)],
                       dstb.at[pl.ds(reg * 24, c)], semB)

    def idx_wait(row_base, c, reg):
      pltpu.make_async_copy(src_hbm.at[pl.ds(row_base, c)],
                            srcb.at[pl.ds(reg * 24, c)], semA).wait()
      pltpu.make_async_copy(dst_hbm.at[pl.ds(row_base, c)],
                            dstb.at[pl.ds(reg * 24, c)], semB).wait()

    @pl.when(wid < 24)
    def _():
      idx_load(wid * 80, 24, 0)

    @pl.when(wid >= 24)
    def _():
      idx_load(wid * 72 + 192, 24, 0)

    def zero_row(i, _):
      for j in range(D // 16):
        rows0[i, pl.ds(j * 16, 16)] = jnp.zeros((16,), jnp.float32)
      return 0
    lax.fori_loop(0, CHUNK, zero_row, 0)

    zrow0 = jnp.minimum(sid * ZROWS, ACC_ROWS - ZROWS)
    n_zc = ZROWS // CHUNK
    zr = ZROWS - n_zc * CHUNK
    for z in range(n_zc):
      pltpu.sync_copy(rows0, acc.at[pl.ds(zrow0 + z * CHUNK, CHUNK)])
    if zr:
      pltpu.sync_copy(rows0.at[pl.ds(0, zr)],
                      acc.at[pl.ds(zrow0 + n_zc * CHUNK, zr)])
    plsc.subcore_barrier()

    def emit_loop(off, c):
      pltpu.async_copy(x_hbm.at[srcb.at[off]], rows0, sem0)
      n_iter = c // 2

      def body(i, _):
        t0 = off + 2 * i
        pltpu.async_copy(x_hbm.at[srcb.at[t0 + 1]], rows1, sem1)
        pltpu.make_async_copy(x_hbm.at[srcb.at[t0]], rows0, sem0).wait()
        pltpu.sync_copy(rows0, acc.at[dstb.at[t0]], add=True)

        @pl.when(i < n_iter - 1)
        def _():
          pltpu.async_copy(x_hbm.at[srcb.at[t0 + 2]], rows0, sem0)

        pltpu.make_async_copy(x_hbm.at[srcb.at[t0 + 1]], rows1, sem1).wait()
        pltpu.sync_copy(rows1, acc.at[dstb.at[t0 + 1]], add=True)
        return 0
      lax.fori_loop(0, n_iter, body, 0)

    @pl.when(wid < 24)
    def _():
      base = wid * 80
      idx_wait(base, 24, 0)
      idx_load(base + 24, 24, 1)
      emit_loop(0, 24)
      idx_wait(base + 24, 24, 1)
      idx_load(base + 48, 16, 0)
      emit_loop(24, 24)
      idx_wait(base + 48, 16, 0)
      idx_load(base + 64, 16, 1)
      emit_loop(0, 16)
      idx_wait(base + 64, 16, 1)
      emit_loop(24, 16)

    @pl.when(wid >= 24)
    def _():
      base = wid * 72 + 192
      idx_wait(base, 24, 0)
      idx_load(base + 24, 24, 1)
      emit_loop(0, 24)
      idx_wait(base + 24, 24, 1)
      idx_load(base + 48, 24, 0)
      emit_loop(24, 24)
      idx_wait(base + 48, 24, 0)
      pltpu.async_copy(tail_hbm.at[0], srcb.at[pl.ds(24, 8)], semA)
      pltpu.async_copy(tail_hbm.at[1], dstb.at[pl.ds(24, 8)], semB)
      emit_loop(0, 24)
      pltpu.make_async_copy(tail_hbm.at[0], srcb.at[pl.ds(24, 8)], semA).wait()
      pltpu.make_async_copy(tail_hbm.at[1], dstb.at[pl.ds(24, 8)], semB).wait()
      t = wid - 24 + 24
      pltpu.sync_copy(x_hbm.at[srcb.at[t]], rows0)
      pltpu.sync_copy(rows0, acc.at[dstb.at[t]], add=True)

    plsc.subcore_barrier()

    crow0 = jnp.minimum(sid * COPY_ROWS, N - COPY_ROWS)
    pltpu.sync_copy(acc.at[pl.ds(crow0, COPY_ROWS)],
                    out_hbm.at[cid, pl.ds(crow0, COPY_ROWS)])

  return sc_agg


_sc_agg = _sc_agg_build()

ROW_BLK = 10000


def _mlp_body(x_ref, agg_ref, w1_ref, b1_ref, w2_ref, b2_ref, o_ref):
  h = x_ref[...] + agg_ref[0] + agg_ref[1]
  h = jnp.dot(h, w1_ref[...], preferred_element_type=jnp.float32) + b1_ref[...]
  h = jnp.where(h > 0, h, jnp.exp(h) - 1.0)
  o_ref[...] = (
      jnp.dot(h, w2_ref[...], preferred_element_type=jnp.float32) + b2_ref[...]
  )


def _mlp(x, agg, W1, b1, W2, b2):
  grid = (N // ROW_BLK,)
  row_spec = pl.BlockSpec((ROW_BLK, D), lambda i: (i, 0))
  agg_spec = pl.BlockSpec((NC, ROW_BLK, D), lambda i: (0, i, 0))
  full_spec = pl.BlockSpec((D, D), lambda i: (0, 0))
  bias_spec = pl.BlockSpec((1, D), lambda i: (0, 0))
  return pl.pallas_call(
      _mlp_body,
      grid=grid,
      in_specs=[row_spec, agg_spec, full_spec, bias_spec,
                full_spec, bias_spec],
      out_specs=row_spec,
      out_shape=jax.ShapeDtypeStruct((N, D), jnp.float32),
  )(x, agg, W1, b1.reshape(1, D), W2, b2.reshape(1, D))


_DUM_SRC = jnp.arange(TAIL, dtype=jnp.int32) % N
_DUM_DST = N + (jnp.arange(TAIL, dtype=jnp.int32) % DUMMY_ROWS)


@jax.jit
def kernel(x, edge_index, W1, b1, W2, b2):
  ei = edge_index.astype(jnp.int32)
  e_main = MAIN_ROWS * CHUNK
  src_main = ei[0, :e_main].reshape(MAIN_ROWS, CHUNK)
  dst_main = ei[1, :e_main].reshape(MAIN_ROWS, CHUNK)
  tail = jnp.stack([
      jnp.concatenate([ei[0, e_main:], _DUM_SRC]).reshape(8, CHUNK),
      jnp.concatenate([ei[1, e_main:], _DUM_DST]).reshape(8, CHUNK),
  ])
  agg2 = _sc_agg(x, src_main, dst_main, tail)
  return _mlp(x, agg2, W1, b1, W2, b2)


# speedup vs baseline: 1.0075x; 1.0075x over previous
import functools

import jax
import jax.numpy as jnp
from jax import lax
from jax.experimental import pallas as pl
from jax.experimental.pallas import tpu as pltpu
from jax.experimental.pallas import tpu_sc as plsc

N = 10000
D = 128
E = 320000

NC = 2
NS = 16
NW = NC * NS

CHUNK = 128
MAIN_ROWS = 2496
TAIL = (E // CHUNK - MAIN_ROWS) * CHUNK
PASS_MAX = 40

DUMMY_ROWS = 64
ACC_ROWS = N + DUMMY_ROWS
ZROWS = 632
COPY_ROWS = 632


def _sc_agg_build():
  mesh = plsc.VectorSubcoreMesh(core_axis_name="c", subcore_axis_name="s")

  @functools.partial(
      pl.kernel,
      mesh=mesh,
      out_type=jax.ShapeDtypeStruct((NC, N, D), jnp.float32),
      scratch_types=[
          pltpu.VMEM((48, CHUNK), jnp.int32),
          pltpu.VMEM((48, CHUNK), jnp.int32),
          pltpu.VMEM((CHUNK, D), jnp.float32),
          pltpu.VMEM((CHUNK, D), jnp.float32),
          pltpu.VMEM_SHARED((ACC_ROWS, D), jnp.float32),
          pltpu.SemaphoreType.DMA,
          pltpu.SemaphoreType.DMA,
          pltpu.SemaphoreType.DMA,
          pltpu.SemaphoreType.DMA,
      ],
  )
  def sc_agg(x_hbm, src_hbm, dst_hbm, tail_hbm, out_hbm, srcb, dstb, rows0,
             rows1, acc, sem0, sem1, semA, semB):
    cid = lax.axis_index("c")
    sid = lax.axis_index("s")
    wid = cid * NS + sid

    def idx_load(row_base, c, reg):
      pltpu.async_copy(src_hbm.at[pl.ds(row_base, c)],
                       srcb.at[pl.ds(reg * 24, c)], semA)
      pltpu.async_copy(dst_hbm.at[pl.ds(row_base, c)],
                       dstb.at[pl.ds(reg * 24, c)], semB)

    def idx_wait(row_base, c, reg):
      pltpu.make_async_copy(src_hbm.at[pl.ds(row_base, c)],
                            srcb.at[pl.ds(reg * 24, c)], semA).wait()
      pltpu.make_async_copy(dst_hbm.at[pl.ds(row_base, c)],
                            dstb.at[pl.ds(reg * 24, c)], semB).wait()

    @pl.when(wid < 24)
    def _():
      idx_load(wid * 80, 24, 0)

    @pl.when(wid >= 24)
    def _():
      idx_load(wid * 72 + 192, 24, 0)

    def zero_row(i, _):
      for j in range(D // 16):
        rows0[i, pl.ds(j * 16, 16)] = jnp.zeros((16,), jnp.float32)
      return 0
    lax.fori_loop(0, CHUNK, zero_row, 0)

    zrow0 = jnp.minimum(sid * ZROWS, ACC_ROWS - ZROWS)
    n_zc = ZROWS // CHUNK
    zr = ZROWS - n_zc * CHUNK
    for z in range(n_zc):
      pltpu.sync_copy(rows0, acc.at[pl.ds(zrow0 + z * CHUNK, CHUNK)])
    if zr:
      pltpu.sync_copy(rows0.at[pl.ds(0, zr)],
                      acc.at[pl.ds(zrow0 + n_zc * CHUNK, zr)])
    plsc.subcore_barrier()

    def emit_loop(off, c):
      pltpu.async_copy(x_hbm.at[srcb.at[off]], rows0, sem0)
      n_iter = c // 2

      def body(i, _):
        t0 = off + 2 * i
        pltpu.async_copy(x_hbm.at[srcb.at[t0 + 1]], rows1, sem1)
        pltpu.make_async_copy(x_hbm.at[srcb.at[t0]], rows0, sem0).wait()
        pltpu.sync_copy(rows0, acc.at[dstb.at[t0]], add=True)

        @pl.when(i < n_iter - 1)
        def _():
          pltpu.async_copy(x_hbm.at[srcb.at[t0 + 2]], rows0, sem0)

        pltpu.make_async_copy(x_hbm.at[srcb.at[t0 + 1]], rows1, sem1).wait()
        pltpu.sync_copy(rows1, acc.at[dstb.at[t0 + 1]], add=True)
        return 0
      lax.fori_loop(0, n_iter, body, 0)

    @pl.when(wid < 24)
    def _():
      base = wid * 80
      idx_wait(base, 24, 0)
      idx_load(base + 24, 24, 1)
      emit_loop(0, 24)
      idx_wait(base + 24, 24, 1)
      idx_load(base + 48, 16, 0)
      emit_loop(24, 24)
      idx_wait(base + 48, 16, 0)
      idx_load(base + 64, 16, 1)
      emit_loop(0, 16)
      idx_wait(base + 64, 16, 1)
      emit_loop(24, 16)

    @pl.when(wid >= 24)
    def _():
      base = wid * 72 + 192
      idx_wait(base, 24, 0)
      idx_load(base + 24, 24, 1)
      emit_loop(0, 24)
      idx_wait(base + 24, 24, 1)
      idx_load(base + 48, 24, 0)
      emit_loop(24, 24)
      idx_wait(base + 48, 24, 0)
      pltpu.async_copy(tail_hbm.at[0], srcb.at[pl.ds(24, 8)], semA)
      pltpu.async_copy(tail_hbm.at[1], dstb.at[pl.ds(24, 8)], semB)
      emit_loop(0, 24)
      pltpu.make_async_copy(tail_hbm.at[0], srcb.at[pl.ds(24, 8)], semA).wait()
      pltpu.make_async_copy(tail_hbm.at[1], dstb.at[pl.ds(24, 8)], semB).wait()
      t = wid - 24 + 24
      pltpu.sync_copy(x_hbm.at[srcb.at[t]], rows0)
      pltpu.sync_copy(rows0, acc.at[dstb.at[t]], add=True)

    plsc.subcore_barrier()

    crow0 = jnp.minimum(sid * COPY_ROWS, N - COPY_ROWS)
    pltpu.sync_copy(acc.at[pl.ds(crow0, COPY_ROWS)],
                    out_hbm.at[cid, pl.ds(crow0, COPY_ROWS)])

  return sc_agg


_sc_agg = _sc_agg_build()

ROW_BLK = 5000


def _mlp_body(x_ref, agg_ref, w1_ref, b1_ref, w2_ref, b2_ref, o_ref):
  h = x_ref[...] + agg_ref[0] + agg_ref[1]
  h = jnp.dot(h, w1_ref[...], preferred_element_type=jnp.float32) + b1_ref[...]
  h = jnp.where(h > 0, h, jnp.exp(h) - 1.0)
  o_ref[...] = (
      jnp.dot(h, w2_ref[...], preferred_element_type=jnp.float32) + b2_ref[...]
  )


def _mlp(x, agg, W1, b1, W2, b2):
  grid = (N // ROW_BLK,)
  row_spec = pl.BlockSpec((ROW_BLK, D), lambda i: (i, 0))
  agg_spec = pl.BlockSpec((NC, ROW_BLK, D), lambda i: (0, i, 0))
  full_spec = pl.BlockSpec((D, D), lambda i: (0, 0))
  bias_spec = pl.BlockSpec((1, D), lambda i: (0, 0))
  return pl.pallas_call(
      _mlp_body,
      grid=grid,
      in_specs=[row_spec, agg_spec, full_spec, bias_spec,
                full_spec, bias_spec],
      out_specs=row_spec,
      out_shape=jax.ShapeDtypeStruct((N, D), jnp.float32),
  )(x, agg, W1, b1.reshape(1, D), W2, b2.reshape(1, D))


_DUM_SRC = jnp.arange(TAIL, dtype=jnp.int32) % N
_DUM_DST = N + (jnp.arange(TAIL, dtype=jnp.int32) % DUMMY_ROWS)


@jax.jit
def kernel(x, edge_index, W1, b1, W2, b2):
  ei = edge_index.astype(jnp.int32)
  e_main = MAIN_ROWS * CHUNK
  src_main = ei[0, :e_main].reshape(MAIN_ROWS, CHUNK)
  dst_main = ei[1, :e_main].reshape(MAIN_ROWS, CHUNK)
  tail = jnp.stack([
      jnp.concatenate([ei[0, e_main:], _DUM_SRC]).reshape(8, CHUNK),
      jnp.concatenate([ei[1, e_main:], _DUM_DST]).reshape(8, CHUNK),
  ])
  agg2 = _sc_agg(x, src_main, dst_main, tail)
  return _mlp(x, agg2, W1, b1, W2, b2)
